# speedup vs baseline: 17.7168x; 17.7168x over previous
import jax
import jax.numpy as jnp
from jax.experimental import pallas as pl

_K_NN = 10
_TQ = 256


def _fused_body(x_ref, txT_ref, w1_ref, b1_ref, w2_ref, b2_ref, w3_ref,
                b3_ref, lns_ref, lnb_ref, xout_ref, idx_ref, d_ref):
    x = x_ref[0]
    h = jax.nn.gelu(jnp.dot(x, w1_ref[:]) + b1_ref[:])
    h = jax.nn.gelu(jnp.dot(h, w2_ref[:]) + b2_ref[:])
    h = jnp.dot(h, w3_ref[:]) + b3_ref[:]
    mu = jnp.mean(h, axis=-1, keepdims=True)
    var = jnp.var(h, axis=-1, keepdims=True)
    xout_ref[0] = (h - mu) / jnp.sqrt(var + 1e-6) * lns_ref[:] + lnb_ref[:]

    txT = txT_ref[0]
    q = x[:, 4:8]
    acc = None
    for d in range(4):
        diff = q[:, d:d + 1] - txT[d:d + 1, :]
        sq = diff * diff
        acc = sq if acc is None else acc + sq

    n_ctx = acc.shape[1]
    col = jax.lax.broadcasted_iota(jnp.int32, acc.shape, 1)
    idx_cols, d_cols = [], []
    d2m = acc
    for _ in range(_K_NN):
        m = jnp.min(d2m, axis=1, keepdims=True)
        pos = jnp.min(jnp.where(d2m == m, col, n_ctx), axis=1,
                      keepdims=True)
        idx_cols.append(pos)
        d_cols.append(jnp.sqrt(jnp.maximum(m, 0.0)))
        d2m = jnp.where(col == pos, jnp.inf, d2m)
    idx_ref[0] = jnp.concatenate(idx_cols, axis=1)
    d_ref[0] = jnp.concatenate(d_cols, axis=1)


def kernel(s_ctx, f_ctx, s_test, embed_obs, W1, b1, W2, b2, W3, b3,
           ln_scale, ln_bias):
    k = _K_NN
    B, Q, d_s = s_test.shape
    K = s_ctx.shape[1]
    d_f = f_ctx.shape[-1]
    n_rows = Q + K

    e0 = jnp.broadcast_to(embed_obs[0], (B, Q, embed_obs.shape[1]))
    e1 = jnp.broadcast_to(embed_obs[1], (B, K, embed_obs.shape[1]))
    f_test = jnp.zeros((B, Q, d_f), f_ctx.dtype)
    X = jnp.concatenate([
        jnp.concatenate([e0, s_test, f_test], axis=-1),
        jnp.concatenate([e1, s_ctx, f_ctx], axis=-1),
    ], axis=1)
    txT = jnp.swapaxes(s_ctx, 1, 2)

    n_tiles = n_rows // _TQ
    grid = (B, n_tiles)
    full = lambda a: pl.BlockSpec(a.shape, lambda b, t: (0,) * a.ndim)
    row_spec = lambda w: pl.BlockSpec((1, _TQ, w), lambda b, t: (b, t, 0))

    x_all, idx_all, d_all = pl.pallas_call(
        _fused_body,
        grid=grid,
        in_specs=[
            row_spec(X.shape[-1]),
            pl.BlockSpec((1, d_s, K), lambda b, t: (b, 0, 0)),
            full(W1), full(b1.reshape(1, -1)), full(W2),
            full(b2.reshape(1, -1)), full(W3), full(b3.reshape(1, -1)),
            full(ln_scale.reshape(1, -1)), full(ln_bias.reshape(1, -1)),
        ],
        out_specs=[row_spec(64), row_spec(k), row_spec(k)],
        out_shape=[
            jax.ShapeDtypeStruct((B, n_rows, 64), jnp.float32),
            jax.ShapeDtypeStruct((B, n_rows, k), jnp.int32),
            jax.ShapeDtypeStruct((B, n_rows, k), jnp.float32),
        ],
    )(X, txT, W1, b1.reshape(1, -1), W2, b2.reshape(1, -1), W3,
      b3.reshape(1, -1), ln_scale.reshape(1, -1), ln_bias.reshape(1, -1))

    nodes_tc = x_all
    nodes_cc = x_all[:, Q:]
    tx_tc = idx_all[:, :Q].reshape(B, Q * k)
    tx_cc = idx_all[:, Q:].reshape(B, K * k)
    d_tc = d_all[:, :Q].reshape(B, Q * k)
    d_cc = d_all[:, Q:].reshape(B, K * k)
    rx_tc = jnp.broadcast_to(jnp.repeat(jnp.arange(Q), k), (B, Q * k))
    rx_cc = jnp.broadcast_to(jnp.repeat(jnp.arange(K), k), (B, K * k))
    return (nodes_tc, d_tc, rx_tc, Q + tx_tc, nodes_cc, d_cc, rx_cc, tx_cc)


# speedup vs baseline: 20.6464x; 1.1654x over previous
import jax
import jax.numpy as jnp
from jax.experimental import pallas as pl

_K_NN = 10
_TQ = 256


def _fused_body(x_ref, txT_ref, w1_ref, b1_ref, w2_ref, b2_ref, w3_ref,
                b3_ref, lns_ref, lnb_ref, xout_ref, idx_ref, d_ref):
    x = x_ref[0]
    h = jax.nn.gelu(jnp.dot(x, w1_ref[:]) + b1_ref[:])
    h = jax.nn.gelu(jnp.dot(h, w2_ref[:]) + b2_ref[:])
    h = jnp.dot(h, w3_ref[:]) + b3_ref[:]
    mu = jnp.mean(h, axis=-1, keepdims=True)
    var = jnp.var(h, axis=-1, keepdims=True)
    xout_ref[0] = (h - mu) / jnp.sqrt(var + 1e-6) * lns_ref[:] + lnb_ref[:]

    txT = txT_ref[0]
    q = x[:, 4:8]
    acc = None
    for d in range(4):
        diff = q[:, d:d + 1] - txT[d:d + 1, :]
        sq = diff * diff
        acc = sq if acc is None else acc + sq

    n_ctx = acc.shape[1]
    colf = jax.lax.broadcasted_iota(jnp.int32, acc.shape, 1).astype(jnp.float32)
    idx_cols, d_cols = [], []
    d2m = acc
    eqmask = None
    for _ in range(_K_NN):
        if eqmask is not None:
            d2m = jnp.where(eqmask, jnp.inf, d2m)
        m = jnp.min(d2m, axis=1, keepdims=True)
        eqmask = d2m == m
        posf = jnp.min(jnp.where(eqmask, colf, float(n_ctx)), axis=1,
                       keepdims=True)
        idx_cols.append(posf.astype(jnp.int32))
        d_cols.append(jnp.sqrt(jnp.maximum(m, 0.0)))
    idx_ref[0] = jnp.concatenate(idx_cols, axis=1)
    d_ref[0] = jnp.concatenate(d_cols, axis=1)


def kernel(s_ctx, f_ctx, s_test, embed_obs, W1, b1, W2, b2, W3, b3,
           ln_scale, ln_bias):
    k = _K_NN
    B, Q, d_s = s_test.shape
    K = s_ctx.shape[1]
    d_f = f_ctx.shape[-1]
    n_rows = Q + K

    e0 = jnp.broadcast_to(embed_obs[0], (B, Q, embed_obs.shape[1]))
    e1 = jnp.broadcast_to(embed_obs[1], (B, K, embed_obs.shape[1]))
    f_test = jnp.zeros((B, Q, d_f), f_ctx.dtype)
    X = jnp.concatenate([
        jnp.concatenate([e0, s_test, f_test], axis=-1),
        jnp.concatenate([e1, s_ctx, f_ctx], axis=-1),
    ], axis=1)
    txT = jnp.swapaxes(s_ctx, 1, 2)

    n_tiles = n_rows // _TQ
    grid = (B, n_tiles)
    full = lambda a: pl.BlockSpec(a.shape, lambda b, t: (0,) * a.ndim)
    row_spec = lambda w: pl.BlockSpec((1, _TQ, w), lambda b, t: (b, t, 0))

    x_all, idx_all, d_all = pl.pallas_call(
        _fused_body,
        grid=grid,
        in_specs=[
            row_spec(X.shape[-1]),
            pl.BlockSpec((1, d_s, K), lambda b, t: (b, 0, 0)),
            full(W1), full(b1.reshape(1, -1)), full(W2),
            full(b2.reshape(1, -1)), full(W3), full(b3.reshape(1, -1)),
            full(ln_scale.reshape(1, -1)), full(ln_bias.reshape(1, -1)),
        ],
        out_specs=[row_spec(64), row_spec(k), row_spec(k)],
        out_shape=[
            jax.ShapeDtypeStruct((B, n_rows, 64), jnp.float32),
            jax.ShapeDtypeStruct((B, n_rows, k), jnp.int32),
            jax.ShapeDtypeStruct((B, n_rows, k), jnp.float32),
        ],
    )(X, txT, W1, b1.reshape(1, -1), W2, b2.reshape(1, -1), W3,
      b3.reshape(1, -1), ln_scale.reshape(1, -1), ln_bias.reshape(1, -1))

    nodes_tc = x_all
    nodes_cc = x_all[:, Q:]
    tx_tc = idx_all[:, :Q].reshape(B, Q * k)
    tx_cc = idx_all[:, Q:].reshape(B, K * k)
    d_tc = d_all[:, :Q].reshape(B, Q * k)
    d_cc = d_all[:, Q:].reshape(B, K * k)
    rx_tc = jnp.broadcast_to(jnp.repeat(jnp.arange(Q), k), (B, Q * k))
    rx_cc = jnp.broadcast_to(jnp.repeat(jnp.arange(K), k), (B, K * k))
    return (nodes_tc, d_tc, rx_tc, Q + tx_tc, nodes_cc, d_cc, rx_cc, tx_cc)
